# Initial kernel scaffold; baseline (speedup 1.0000x reference)
import jax
import jax.numpy as jnp
from jax.experimental import pallas as pl


def kernel(x, W_h, b_h, W_g, b_g, W_r, b_r, M, W_o, b_o):
    raise NotImplementedError("write your pallas kernel here")


# speedup vs baseline: 3.6513x; 3.6513x over previous
import jax
import jax.numpy as jnp
from jax.experimental import pallas as pl
from jax.experimental.pallas import tpu as pltpu

_BLK = 2048


def _ctrl_kernel(x_ref, xl_ref, Wh_ref, bh_ref, Wg_ref, bg_ref, Wr_ref,
                 br_ref, h_ref, rk_ref, gate_ref):
    x = x_ref[...]
    h = jnp.tanh(jnp.dot(x, Wh_ref[...], preferred_element_type=jnp.float32)
                 + bh_ref[...])
    h_ref[...] = h
    rk = jnp.dot(h, Wr_ref[...], preferred_element_type=jnp.float32) + br_ref[...]
    knorm = jnp.sqrt(jnp.sum(rk * rk, axis=1, keepdims=True)) + 1e-8
    rk_ref[...] = rk / knorm
    gate_ref[...] = (jnp.dot(xl_ref[...], Wg_ref[...],
                             preferred_element_type=jnp.float32) + bg_ref[...])


def _flash_kernel(rk_ref, M_ref, r_ref, siml_ref, l_ref, acc_ref, lsum_ref):
    i = pl.program_id(0)
    nb = pl.num_programs(0)

    @pl.when(i == 0)
    def _init():
        acc_ref[...] = jnp.zeros_like(acc_ref)
        lsum_ref[...] = jnp.zeros_like(lsum_ref)

    Mb = M_ref[...]
    msq = jnp.sum(Mb * Mb, axis=1, keepdims=True)
    Mn = Mb / (jnp.sqrt(msq) + 1e-8)
    sim = jax.lax.dot_general(rk_ref[...], Mn, (((1,), (1,)), ((), ())),
                              preferred_element_type=jnp.float32)
    p = jnp.exp(sim)
    lsum_ref[...] += jnp.sum(p, axis=1, keepdims=True)
    acc_ref[...] += jnp.dot(p, Mb, preferred_element_type=jnp.float32)
    siml_ref[...] = sim[-1:, :]

    @pl.when(i == nb - 1)
    def _fin():
        l = lsum_ref[...]
        r_ref[...] = acc_ref[...] / l
        l_ref[...] = l


def _final_kernel(h_ref, r_ref, Wo_ref, bo_ref, siml_ref, llast_ref,
                  out_ref, w_ref):
    cd = h_ref.shape[1]
    Wo = Wo_ref[...]
    out = (jnp.dot(h_ref[...], Wo[:cd, :], preferred_element_type=jnp.float32)
           + jnp.dot(r_ref[...], Wo[cd:, :], preferred_element_type=jnp.float32)
           + bo_ref[...])
    out_ref[...] = out
    w_ref[...] = jnp.exp(siml_ref[...]) / llast_ref[...]


def kernel(x, W_h, b_h, W_g, b_g, W_r, b_r, M, W_o, b_o):
    B, _ = x.shape
    CD = W_h.shape[1]
    L, LS = M.shape
    nb = L // _BLK

    bh2 = b_h.reshape(1, CD)
    bg2 = b_g.reshape(1, 1)
    br2 = b_r.reshape(1, LS)
    bo2 = b_o.reshape(1, 1)
    x_last = x[B - 1:B, :]

    h, rk, gate = pl.pallas_call(
        _ctrl_kernel,
        out_shape=(
            jax.ShapeDtypeStruct((B, CD), jnp.float32),
            jax.ShapeDtypeStruct((B, LS), jnp.float32),
            jax.ShapeDtypeStruct((1, 1), jnp.float32),
        ),
    )(x, x_last, W_h, bh2, W_g, bg2, W_r, br2)

    r, siml, l = pl.pallas_call(
        _flash_kernel,
        grid=(nb,),
        in_specs=[
            pl.BlockSpec((B, LS), lambda i: (0, 0)),
            pl.BlockSpec((_BLK, LS), lambda i: (i, 0)),
        ],
        out_specs=[
            pl.BlockSpec((B, LS), lambda i: (0, 0)),
            pl.BlockSpec((1, _BLK), lambda i: (0, i)),
            pl.BlockSpec((B, 1), lambda i: (0, 0)),
        ],
        out_shape=(
            jax.ShapeDtypeStruct((B, LS), jnp.float32),
            jax.ShapeDtypeStruct((1, L), jnp.float32),
            jax.ShapeDtypeStruct((B, 1), jnp.float32),
        ),
        scratch_shapes=[
            pltpu.VMEM((B, LS), jnp.float32),
            pltpu.VMEM((B, 1), jnp.float32),
        ],
    )(rk, M)

    out, w = pl.pallas_call(
        _final_kernel,
        out_shape=(
            jax.ShapeDtypeStruct((B, 1), jnp.float32),
            jax.ShapeDtypeStruct((1, L), jnp.float32),
        ),
    )(h, r, W_o, bo2, siml, l[B - 1:B, :])

    return (out[:, 0], h[B - 1], gate[0], w[0])


# speedup vs baseline: 3.7684x; 1.0321x over previous
import jax
import jax.numpy as jnp
from jax.experimental import pallas as pl
from jax.experimental.pallas import tpu as pltpu

_BLK = 2048


def _ctrl_kernel(x_ref, xl_ref, Wh_ref, bh_ref, Wg_ref, bg_ref, Wr_ref,
                 br_ref, h_ref, rk_ref, gate_ref):
    x = x_ref[...]
    h = jnp.tanh(jnp.dot(x, Wh_ref[...], preferred_element_type=jnp.float32)
                 + bh_ref[...])
    h_ref[...] = h
    rk = jnp.dot(h, Wr_ref[...], preferred_element_type=jnp.float32) + br_ref[...]
    knorm = jnp.sqrt(jnp.sum(rk * rk, axis=1, keepdims=True)) + 1e-8
    rk_ref[...] = rk / knorm
    gate_ref[...] = (jnp.dot(xl_ref[...], Wg_ref[...],
                             preferred_element_type=jnp.float32) + bg_ref[...])


def _flash_kernel(rk_ref, M_ref, r_ref, siml_ref, l_ref, acc_ref, lsum_ref):
    i = pl.program_id(0)
    nb = pl.num_programs(0)

    @pl.when(i == 0)
    def _init():
        acc_ref[...] = jnp.zeros_like(acc_ref)
        lsum_ref[...] = jnp.zeros_like(lsum_ref)

    Mb = M_ref[...]
    msq = jnp.sum(Mb * Mb, axis=1, keepdims=True)
    Mn = Mb / (jnp.sqrt(msq) + 1e-8)
    sim = jax.lax.dot_general(rk_ref[...], Mn, (((1,), (1,)), ((), ())),
                              preferred_element_type=jnp.float32)
    p = jnp.exp(sim)
    lsum_ref[...] += jnp.sum(p, axis=1, keepdims=True)
    acc_ref[...] += jnp.dot(p.astype(jnp.bfloat16), Mb.astype(jnp.bfloat16),
                            preferred_element_type=jnp.float32)
    siml_ref[...] = sim[-1:, :]

    @pl.when(i == nb - 1)
    def _fin():
        l = lsum_ref[...]
        r_ref[...] = acc_ref[...] / l
        l_ref[...] = l


def _final_kernel(h_ref, r_ref, Wo_ref, bo_ref, siml_ref, llast_ref,
                  out_ref, w_ref):
    cd = h_ref.shape[1]
    Wo = Wo_ref[...]
    out = (jnp.dot(h_ref[...], Wo[:cd, :], preferred_element_type=jnp.float32)
           + jnp.dot(r_ref[...], Wo[cd:, :], preferred_element_type=jnp.float32)
           + bo_ref[...])
    out_ref[...] = out
    w_ref[...] = jnp.exp(siml_ref[...]) / llast_ref[...]


def kernel(x, W_h, b_h, W_g, b_g, W_r, b_r, M, W_o, b_o):
    B, _ = x.shape
    CD = W_h.shape[1]
    L, LS = M.shape
    nb = L // _BLK

    bh2 = b_h.reshape(1, CD)
    bg2 = b_g.reshape(1, 1)
    br2 = b_r.reshape(1, LS)
    bo2 = b_o.reshape(1, 1)
    x_last = x[B - 1:B, :]

    h, rk, gate = pl.pallas_call(
        _ctrl_kernel,
        out_shape=(
            jax.ShapeDtypeStruct((B, CD), jnp.float32),
            jax.ShapeDtypeStruct((B, LS), jnp.float32),
            jax.ShapeDtypeStruct((1, 1), jnp.float32),
        ),
    )(x, x_last, W_h, bh2, W_g, bg2, W_r, br2)

    r, siml, l = pl.pallas_call(
        _flash_kernel,
        grid=(nb,),
        in_specs=[
            pl.BlockSpec((B, LS), lambda i: (0, 0)),
            pl.BlockSpec((_BLK, LS), lambda i: (i, 0)),
        ],
        out_specs=[
            pl.BlockSpec((B, LS), lambda i: (0, 0)),
            pl.BlockSpec((1, _BLK), lambda i: (0, i)),
            pl.BlockSpec((B, 1), lambda i: (0, 0)),
        ],
        out_shape=(
            jax.ShapeDtypeStruct((B, LS), jnp.float32),
            jax.ShapeDtypeStruct((1, L), jnp.float32),
            jax.ShapeDtypeStruct((B, 1), jnp.float32),
        ),
        scratch_shapes=[
            pltpu.VMEM((B, LS), jnp.float32),
            pltpu.VMEM((B, 1), jnp.float32),
        ],
    )(rk, M)

    out, w = pl.pallas_call(
        _final_kernel,
        out_shape=(
            jax.ShapeDtypeStruct((B, 1), jnp.float32),
            jax.ShapeDtypeStruct((1, L), jnp.float32),
        ),
    )(h, r, W_o, bo2, siml, l[B - 1:B, :])

    return (out[:, 0], h[B - 1], gate[0], w[0])
